# Initial kernel scaffold; baseline (speedup 1.0000x reference)
import jax
import jax.numpy as jnp
from jax.experimental import pallas as pl


def kernel(x, edge_index, w_r_W, w_r_b, w_z_W, w_z_b, w_h_W, w_h_b, gcn_W, gcn_b):
    raise NotImplementedError("write your pallas kernel here")


# speedup vs baseline: 5.7827x; 5.7827x over previous
import functools

import jax
import jax.numpy as jnp
from jax import lax
from jax.experimental import pallas as pl
from jax.experimental.pallas import tpu as pltpu
from jax.experimental.pallas import tpu_sc as plsc

N = 10000
E = 320000
H = 128
SEQ = 8

NUM_CORES = 2
NUM_SUBCORES = 16
NUM_TILES = NUM_CORES * NUM_SUBCORES
EDGES_PER_TILE = E // NUM_TILES
CHUNK = 80
N_CHUNKS = EDGES_PER_TILE // CHUNK
NPAD = 10240
ROWS_PER_TILE = NPAD // NUM_SUBCORES
ZROWS = 128

_sc_mesh = plsc.VectorSubcoreMesh(core_axis_name="c", subcore_axis_name="s")


@functools.partial(
    pl.kernel,
    out_type=jax.ShapeDtypeStruct((NUM_CORES, NPAD, H), jnp.float32),
    mesh=_sc_mesh,
    scratch_types=[
        pltpu.VMEM((CHUNK,), jnp.int32),
        pltpu.VMEM((CHUNK, H), jnp.float32),
        pltpu.VMEM((ZROWS, H), jnp.float32),
        pltpu.VMEM_SHARED((NPAD, H), jnp.float32),
    ],
)
def _degree_kernel(src_hbm, dst_hbm, out_hbm, idx_v, ones_v, zero_v, cnt_sh):
    cid = lax.axis_index("c")
    sid = lax.axis_index("s")
    row0 = sid * ROWS_PER_TILE

    def fill(i, _):
        r = i // 8
        c = (i % 8) * 16
        ones_v[r, pl.ds(c, 16)] = jnp.full((16,), 1.0, jnp.float32)
        return 0

    lax.fori_loop(0, CHUNK * 8, fill, 0)

    def zfill(i, _):
        r = i // 8
        c = (i % 8) * 16
        zero_v[r, pl.ds(c, 16)] = jnp.zeros((16,), jnp.float32)
        return 0

    lax.fori_loop(0, ZROWS * 8, zfill, 0)

    def zcopy(k, _):
        pltpu.sync_copy(zero_v, cnt_sh.at[pl.ds(row0 + k * ZROWS, ZROWS)])
        return 0

    lax.fori_loop(0, ROWS_PER_TILE // ZROWS, zcopy, 0)
    plsc.subcore_barrier()

    epc = E // NUM_SUBCORES
    base0 = sid * epc

    def count(idx_hbm):
        def body(j, _):
            base = base0 + j * CHUNK
            pltpu.sync_copy(idx_hbm.at[pl.ds(base, CHUNK)], idx_v)
            pltpu.sync_copy(ones_v, cnt_sh.at[idx_v], add=True)
            return 0

        lax.fori_loop(0, epc // CHUNK, body, 0)

    @pl.when(cid == 0)
    def _():
        count(src_hbm)

    @pl.when(cid == 1)
    def _():
        count(dst_hbm)

    plsc.subcore_barrier()
    pltpu.sync_copy(cnt_sh.at[pl.ds(row0, ROWS_PER_TILE)],
                    out_hbm.at[cid, pl.ds(row0, ROWS_PER_TILE)])


@functools.partial(
    pl.kernel,
    out_type=jax.ShapeDtypeStruct((NUM_CORES, NPAD, H), jnp.float32),
    mesh=_sc_mesh,
    scratch_types=[
        pltpu.VMEM((CHUNK,), jnp.int32),
        pltpu.VMEM((CHUNK,), jnp.int32),
        pltpu.VMEM((CHUNK, H), jnp.float32),
        pltpu.VMEM((ZROWS, H), jnp.float32),
        pltpu.VMEM_SHARED((NPAD, H), jnp.float32),
        pltpu.SemaphoreType.DMA,
    ],
)
def _edge_agg_kernel(p_hbm, src_hbm, dst_hbm, out_hbm, src_v, dst_v, rows_v,
                     zero_v, agg_sh, sem):
    cid = lax.axis_index("c")
    sid = lax.axis_index("s")
    wid = sid * NUM_CORES + cid
    row0 = sid * ROWS_PER_TILE

    def zfill(i, _):
        r = i // 8
        c = (i % 8) * 16
        zero_v[r, pl.ds(c, 16)] = jnp.zeros((16,), jnp.float32)
        return 0

    lax.fori_loop(0, ZROWS * 8, zfill, 0)

    def zcopy(k, _):
        pltpu.sync_copy(zero_v, agg_sh.at[pl.ds(row0 + k * ZROWS, ZROWS)])
        return 0

    lax.fori_loop(0, ROWS_PER_TILE // ZROWS, zcopy, 0)
    plsc.subcore_barrier()

    base0 = wid * EDGES_PER_TILE

    def body(j, _):
        base = base0 + j * CHUNK
        pltpu.sync_copy(src_hbm.at[pl.ds(base, CHUNK)], src_v)
        pltpu.sync_copy(dst_hbm.at[pl.ds(base, CHUNK)], dst_v)
        pltpu.async_copy(p_hbm.at[src_v], rows_v, sem).wait()
        pltpu.sync_copy(rows_v, agg_sh.at[dst_v], add=True)
        return 0

    lax.fori_loop(0, N_CHUNKS, body, 0)
    plsc.subcore_barrier()
    pltpu.sync_copy(agg_sh.at[pl.ds(row0, ROWS_PER_TILE)],
                    out_hbm.at[cid, pl.ds(row0, ROWS_PER_TILE)])


BLK = 1000
GRID = N // BLK


def _setup_body(cnt, x, wr, br, wz, bz, wh, bh,
                inv_out_o, inv_in_o, xr_o, xz_o, xh_o):
    c = cnt[...]
    deg_out = c[0, :, 0]
    deg_in = c[1, :, 0]
    inv_out_o[...] = jnp.where(deg_out > 0, lax.rsqrt(deg_out), 0.0)[:, None]
    inv_in_o[...] = jnp.where(deg_in > 0, lax.rsqrt(deg_in), 0.0)[:, None]
    xr_o[...] = x[...] @ wr[...] + br[...]
    xz_o[...] = x[...] @ wz[...] + bz[...]
    xh_o[...] = x[...] @ wh[...] + bh[...]


_setup_call = pl.pallas_call(
    _setup_body,
    grid=(GRID,),
    in_specs=[
        pl.BlockSpec((2, BLK, H), lambda i: (0, i, 0)),
        pl.BlockSpec((1, H), lambda i: (0, 0)),
        pl.BlockSpec((H, H), lambda i: (0, 0)),
        pl.BlockSpec((1, H), lambda i: (0, 0)),
        pl.BlockSpec((H, H), lambda i: (0, 0)),
        pl.BlockSpec((1, H), lambda i: (0, 0)),
        pl.BlockSpec((H, H), lambda i: (0, 0)),
        pl.BlockSpec((1, H), lambda i: (0, 0)),
    ],
    out_specs=[
        pl.BlockSpec((BLK, 1), lambda i: (i, 0)),
        pl.BlockSpec((BLK, 1), lambda i: (i, 0)),
        pl.BlockSpec((1, H), lambda i: (0, 0)),
        pl.BlockSpec((1, H), lambda i: (0, 0)),
        pl.BlockSpec((1, H), lambda i: (0, 0)),
    ],
    out_shape=[
        jax.ShapeDtypeStruct((N, 1), jnp.float32),
        jax.ShapeDtypeStruct((N, 1), jnp.float32),
        jax.ShapeDtypeStruct((1, H), jnp.float32),
        jax.ShapeDtypeStruct((1, H), jnp.float32),
        jax.ShapeDtypeStruct((1, H), jnp.float32),
    ],
)


def _step1_body(inv_out, xr, xz, xh, gb, gw, h_o, p_o):
    gh = gb[...]
    r = jax.nn.sigmoid(xr[...] + gh)
    z = jax.nn.sigmoid(xz[...] + gh)
    ht = jnp.tanh(xh[...] + r * gh)
    h1 = z * ht
    h_o[...] = jnp.broadcast_to(h1, (BLK, H))
    p_o[...] = inv_out[...] * jnp.dot(h1, gw[...],
                                      preferred_element_type=jnp.float32)


_step1_call = pl.pallas_call(
    _step1_body,
    grid=(GRID,),
    in_specs=[
        pl.BlockSpec((BLK, 1), lambda i: (i, 0)),
        pl.BlockSpec((1, H), lambda i: (0, 0)),
        pl.BlockSpec((1, H), lambda i: (0, 0)),
        pl.BlockSpec((1, H), lambda i: (0, 0)),
        pl.BlockSpec((1, H), lambda i: (0, 0)),
        pl.BlockSpec((H, H), lambda i: (0, 0)),
    ],
    out_specs=[
        pl.BlockSpec((BLK, H), lambda i: (i, 0)),
        pl.BlockSpec((BLK, H), lambda i: (i, 0)),
    ],
    out_shape=[
        jax.ShapeDtypeStruct((N, H), jnp.float32),
        jax.ShapeDtypeStruct((N, H), jnp.float32),
    ],
)


def _step_body(h, agg, inv_in, inv_out, xr, xz, xh, gb, gw, h_o, p_o):
    a = agg[...]
    gh = (a[0] + a[1]) * inv_in[...] + gb[...]
    r = jax.nn.sigmoid(xr[...] + gh)
    z = jax.nn.sigmoid(xz[...] + gh)
    ht = jnp.tanh(xh[...] + r * gh)
    hn = (1.0 - z) * h[...] + z * ht
    h_o[...] = hn
    p_o[...] = jnp.dot(hn * inv_out[...], gw[...],
                       preferred_element_type=jnp.float32)


_step_call = pl.pallas_call(
    _step_body,
    grid=(GRID,),
    in_specs=[
        pl.BlockSpec((BLK, H), lambda i: (i, 0)),
        pl.BlockSpec((2, BLK, H), lambda i: (0, i, 0)),
        pl.BlockSpec((BLK, 1), lambda i: (i, 0)),
        pl.BlockSpec((BLK, 1), lambda i: (i, 0)),
        pl.BlockSpec((1, H), lambda i: (0, 0)),
        pl.BlockSpec((1, H), lambda i: (0, 0)),
        pl.BlockSpec((1, H), lambda i: (0, 0)),
        pl.BlockSpec((1, H), lambda i: (0, 0)),
        pl.BlockSpec((H, H), lambda i: (0, 0)),
    ],
    out_specs=[
        pl.BlockSpec((BLK, H), lambda i: (i, 0)),
        pl.BlockSpec((BLK, H), lambda i: (i, 0)),
    ],
    out_shape=[
        jax.ShapeDtypeStruct((N, H), jnp.float32),
        jax.ShapeDtypeStruct((N, H), jnp.float32),
    ],
)


def kernel(x, edge_index, w_r_W, w_r_b, w_z_W, w_z_b, w_h_W, w_h_b, gcn_W,
           gcn_b):
    x2 = x.reshape(1, H)
    br = w_r_b.reshape(1, H)
    bz = w_z_b.reshape(1, H)
    bh = w_h_b.reshape(1, H)
    gb = gcn_b.reshape(1, H)

    src = edge_index[0]
    dst = edge_index[1]
    cnt = _degree_kernel(src, dst)
    inv_out, inv_in, xr, xz, xh = _setup_call(
        cnt, x2, w_r_W, br, w_z_W, bz, w_h_W, bh)

    h, p = _step1_call(inv_out, xr, xz, xh, gb, gcn_W)
    outs = [h]
    for _ in range(1, SEQ):
        agg = _edge_agg_kernel(p, src, dst)
        h, p = _step_call(h, agg, inv_in, inv_out, xr, xz, xh, gb, gcn_W)
        outs.append(h)
    return jnp.stack(outs, axis=0)[None]


# speedup vs baseline: 12.2411x; 2.1168x over previous
import functools

import jax
import jax.numpy as jnp
from jax import lax
from jax.experimental import pallas as pl
from jax.experimental.pallas import tpu as pltpu
from jax.experimental.pallas import tpu_sc as plsc

N = 10000
E = 320000
H = 128
SEQ = 8

NUM_CORES = 2
NUM_SUBCORES = 16
NUM_TILES = NUM_CORES * NUM_SUBCORES
EDGES_PER_TILE = E // NUM_TILES
CHUNK = 80
N_CHUNKS = EDGES_PER_TILE // CHUNK
NPAD = 10240
ROWS_PER_TILE = NPAD // NUM_SUBCORES
ZROWS = 32

_sc_mesh = plsc.VectorSubcoreMesh(core_axis_name="c", subcore_axis_name="s")


@functools.partial(
    pl.kernel,
    out_type=jax.ShapeDtypeStruct((NUM_CORES, NPAD, H), jnp.float32),
    mesh=_sc_mesh,
    scratch_types=[
        pltpu.VMEM((CHUNK,), jnp.int32),
        pltpu.VMEM((CHUNK, H), jnp.float32),
        pltpu.VMEM((ZROWS, H), jnp.float32),
        pltpu.VMEM_SHARED((NPAD, H), jnp.float32),
    ],
)
def _degree_kernel(src_hbm, dst_hbm, out_hbm, idx_v, ones_v, zero_v, cnt_sh):
    cid = lax.axis_index("c")
    sid = lax.axis_index("s")
    row0 = sid * ROWS_PER_TILE

    def fill(i, _):
        r = i // 8
        c = (i % 8) * 16
        ones_v[r, pl.ds(c, 16)] = jnp.full((16,), 1.0, jnp.float32)
        return 0

    lax.fori_loop(0, CHUNK * 8, fill, 0)

    def zfill(i, _):
        r = i // 8
        c = (i % 8) * 16
        zero_v[r, pl.ds(c, 16)] = jnp.zeros((16,), jnp.float32)
        return 0

    lax.fori_loop(0, ZROWS * 8, zfill, 0)

    def zcopy(k, _):
        pltpu.sync_copy(zero_v, cnt_sh.at[pl.ds(row0 + k * ZROWS, ZROWS)])
        return 0

    lax.fori_loop(0, ROWS_PER_TILE // ZROWS, zcopy, 0)
    plsc.subcore_barrier()

    epc = E // NUM_SUBCORES
    base0 = sid * epc

    def count(idx_hbm):
        def body(j, _):
            base = base0 + j * CHUNK
            pltpu.sync_copy(idx_hbm.at[pl.ds(base, CHUNK)], idx_v)
            pltpu.sync_copy(ones_v, cnt_sh.at[idx_v], add=True)
            return 0

        lax.fori_loop(0, epc // CHUNK, body, 0)

    @pl.when(cid == 0)
    def _():
        count(src_hbm)

    @pl.when(cid == 1)
    def _():
        count(dst_hbm)

    plsc.subcore_barrier()
    pltpu.sync_copy(cnt_sh.at[pl.ds(row0, ROWS_PER_TILE)],
                    out_hbm.at[cid, pl.ds(row0, ROWS_PER_TILE)])


SLOTS = 4


@functools.partial(
    pl.kernel,
    out_type=jax.ShapeDtypeStruct((NUM_CORES, NPAD, H), jnp.float32),
    mesh=_sc_mesh,
    scratch_types=[
        [pltpu.VMEM((CHUNK,), jnp.int32) for _ in range(SLOTS)],
        [pltpu.VMEM((CHUNK,), jnp.int32) for _ in range(SLOTS)],
        [pltpu.VMEM((CHUNK, H), jnp.float32) for _ in range(SLOTS)],
        pltpu.VMEM((ZROWS, H), jnp.float32),
        pltpu.VMEM_SHARED((NPAD, H), jnp.float32),
        [pltpu.SemaphoreType.DMA for _ in range(SLOTS)],
        [pltpu.SemaphoreType.DMA for _ in range(SLOTS)],
    ],
)
def _edge_agg_kernel(p_hbm, src_hbm, dst_hbm, out_hbm, src_v, dst_v, rows,
                     zero_v, agg_sh, isems, gsems):
    cid = lax.axis_index("c")
    sid = lax.axis_index("s")
    wid = sid * NUM_CORES + cid
    row0 = sid * ROWS_PER_TILE
    base0 = wid * EDGES_PER_TILE

    def idx_issue(c, s):
        @pl.when(c < N_CHUNKS)
        def _():
            base = base0 + c * CHUNK
            pltpu.async_copy(src_hbm.at[pl.ds(base, CHUNK)], src_v[s],
                             isems[s])
            pltpu.async_copy(dst_hbm.at[pl.ds(base, CHUNK)], dst_v[s],
                             isems[s])

    def gather_issue(c, s):
        @pl.when(c < N_CHUNKS)
        def _():
            base = base0 + c * CHUNK
            pltpu.make_async_copy(src_hbm.at[pl.ds(base, CHUNK)], src_v[s],
                                  isems[s]).wait()
            pltpu.make_async_copy(dst_hbm.at[pl.ds(base, CHUNK)], dst_v[s],
                                  isems[s]).wait()
            pltpu.async_copy(p_hbm.at[src_v[s]], rows[s], gsems[s])

    def drain(c, s):
        @pl.when(c < N_CHUNKS)
        def _():
            pltpu.make_async_copy(p_hbm.at[src_v[s]], rows[s],
                                  gsems[s]).wait()
            pltpu.sync_copy(rows[s], agg_sh.at[dst_v[s]], add=True)

    for k in range(SLOTS):
        idx_issue(k, k)

    def zfill(i, _):
        r = i // 8
        c = (i % 8) * 16
        zero_v[r, pl.ds(c, 16)] = jnp.zeros((16,), jnp.float32)
        return 0

    lax.fori_loop(0, ZROWS * 8, zfill, 0)

    def zcopy(k, _):
        pltpu.sync_copy(zero_v, agg_sh.at[pl.ds(row0 + k * ZROWS, ZROWS)])
        return 0

    lax.fori_loop(0, ROWS_PER_TILE // ZROWS, zcopy, 0)
    plsc.subcore_barrier()

    gather_issue(0, 0)
    gather_issue(1, 1)

    def body(i, _):
        c0 = 4 * i
        for k in range(SLOTS):
            c = c0 + k
            drain(c, k)
            idx_issue(c + SLOTS, k)
            gather_issue(c + 2, (k + 2) % SLOTS)
        return 0

    lax.fori_loop(0, (N_CHUNKS + SLOTS - 1) // SLOTS, body, 0)
    plsc.subcore_barrier()
    pltpu.sync_copy(agg_sh.at[pl.ds(row0, ROWS_PER_TILE)],
                    out_hbm.at[cid, pl.ds(row0, ROWS_PER_TILE)])


BLK = 1000
GRID = N // BLK


def _setup_body(cnt, x, wr, br, wz, bz, wh, bh,
                inv_out_o, inv_in_o, xr_o, xz_o, xh_o):
    c = cnt[...]
    deg_out = c[0, :, 0]
    deg_in = c[1, :, 0]
    inv_out_o[...] = jnp.where(deg_out > 0, lax.rsqrt(deg_out), 0.0)[:, None]
    inv_in_o[...] = jnp.where(deg_in > 0, lax.rsqrt(deg_in), 0.0)[:, None]
    xr_o[...] = x[...] @ wr[...] + br[...]
    xz_o[...] = x[...] @ wz[...] + bz[...]
    xh_o[...] = x[...] @ wh[...] + bh[...]


_setup_call = pl.pallas_call(
    _setup_body,
    grid=(GRID,),
    in_specs=[
        pl.BlockSpec((2, BLK, H), lambda i: (0, i, 0)),
        pl.BlockSpec((1, H), lambda i: (0, 0)),
        pl.BlockSpec((H, H), lambda i: (0, 0)),
        pl.BlockSpec((1, H), lambda i: (0, 0)),
        pl.BlockSpec((H, H), lambda i: (0, 0)),
        pl.BlockSpec((1, H), lambda i: (0, 0)),
        pl.BlockSpec((H, H), lambda i: (0, 0)),
        pl.BlockSpec((1, H), lambda i: (0, 0)),
    ],
    out_specs=[
        pl.BlockSpec((BLK, 1), lambda i: (i, 0)),
        pl.BlockSpec((BLK, 1), lambda i: (i, 0)),
        pl.BlockSpec((1, H), lambda i: (0, 0)),
        pl.BlockSpec((1, H), lambda i: (0, 0)),
        pl.BlockSpec((1, H), lambda i: (0, 0)),
    ],
    out_shape=[
        jax.ShapeDtypeStruct((N, 1), jnp.float32),
        jax.ShapeDtypeStruct((N, 1), jnp.float32),
        jax.ShapeDtypeStruct((1, H), jnp.float32),
        jax.ShapeDtypeStruct((1, H), jnp.float32),
        jax.ShapeDtypeStruct((1, H), jnp.float32),
    ],
)


def _step1_body(inv_out, xr, xz, xh, gb, gw, h_o, p_o):
    gh = gb[...]
    r = jax.nn.sigmoid(xr[...] + gh)
    z = jax.nn.sigmoid(xz[...] + gh)
    ht = jnp.tanh(xh[...] + r * gh)
    h1 = z * ht
    h_o[...] = jnp.broadcast_to(h1, (BLK, H))
    p_o[...] = inv_out[...] * jnp.dot(h1, gw[...],
                                      preferred_element_type=jnp.float32)


_step1_call = pl.pallas_call(
    _step1_body,
    grid=(GRID,),
    in_specs=[
        pl.BlockSpec((BLK, 1), lambda i: (i, 0)),
        pl.BlockSpec((1, H), lambda i: (0, 0)),
        pl.BlockSpec((1, H), lambda i: (0, 0)),
        pl.BlockSpec((1, H), lambda i: (0, 0)),
        pl.BlockSpec((1, H), lambda i: (0, 0)),
        pl.BlockSpec((H, H), lambda i: (0, 0)),
    ],
    out_specs=[
        pl.BlockSpec((BLK, H), lambda i: (i, 0)),
        pl.BlockSpec((BLK, H), lambda i: (i, 0)),
    ],
    out_shape=[
        jax.ShapeDtypeStruct((N, H), jnp.float32),
        jax.ShapeDtypeStruct((N, H), jnp.float32),
    ],
)


def _step_body(h, agg, inv_in, inv_out, xr, xz, xh, gb, gw, h_o, p_o):
    a = agg[...]
    gh = (a[0] + a[1]) * inv_in[...] + gb[...]
    r = jax.nn.sigmoid(xr[...] + gh)
    z = jax.nn.sigmoid(xz[...] + gh)
    ht = jnp.tanh(xh[...] + r * gh)
    hn = (1.0 - z) * h[...] + z * ht
    h_o[...] = hn
    p_o[...] = jnp.dot(hn * inv_out[...], gw[...],
                       preferred_element_type=jnp.float32)


_step_call = pl.pallas_call(
    _step_body,
    grid=(GRID,),
    in_specs=[
        pl.BlockSpec((BLK, H), lambda i: (i, 0)),
        pl.BlockSpec((2, BLK, H), lambda i: (0, i, 0)),
        pl.BlockSpec((BLK, 1), lambda i: (i, 0)),
        pl.BlockSpec((BLK, 1), lambda i: (i, 0)),
        pl.BlockSpec((1, H), lambda i: (0, 0)),
        pl.BlockSpec((1, H), lambda i: (0, 0)),
        pl.BlockSpec((1, H), lambda i: (0, 0)),
        pl.BlockSpec((1, H), lambda i: (0, 0)),
        pl.BlockSpec((H, H), lambda i: (0, 0)),
    ],
    out_specs=[
        pl.BlockSpec((BLK, H), lambda i: (i, 0)),
        pl.BlockSpec((BLK, H), lambda i: (i, 0)),
    ],
    out_shape=[
        jax.ShapeDtypeStruct((N, H), jnp.float32),
        jax.ShapeDtypeStruct((N, H), jnp.float32),
    ],
)


def kernel(x, edge_index, w_r_W, w_r_b, w_z_W, w_z_b, w_h_W, w_h_b, gcn_W,
           gcn_b):
    x2 = x.reshape(1, H)
    br = w_r_b.reshape(1, H)
    bz = w_z_b.reshape(1, H)
    bh = w_h_b.reshape(1, H)
    gb = gcn_b.reshape(1, H)

    src = edge_index[0]
    dst = edge_index[1]
    cnt = _degree_kernel(src, dst)
    inv_out, inv_in, xr, xz, xh = _setup_call(
        cnt, x2, w_r_W, br, w_z_W, bz, w_h_W, bh)

    h, p = _step1_call(inv_out, xr, xz, xh, gb, gcn_W)
    outs = [h]
    for _ in range(1, SEQ):
        agg = _edge_agg_kernel(p, src, dst)
        h, p = _step_call(h, agg, inv_in, inv_out, xr, xz, xh, gb, gcn_W)
        outs.append(h)
    return jnp.stack(outs, axis=0)[None]


# speedup vs baseline: 13.8539x; 1.1318x over previous
import functools

import jax
import jax.numpy as jnp
from jax import lax
from jax.experimental import pallas as pl
from jax.experimental.pallas import tpu as pltpu
from jax.experimental.pallas import tpu_sc as plsc

N = 10000
E = 320000
H = 128
SEQ = 8

NUM_CORES = 2
NUM_SUBCORES = 16
NUM_TILES = NUM_CORES * NUM_SUBCORES
EDGES_PER_TILE = E // NUM_TILES
CHUNK = 80
N_CHUNKS = EDGES_PER_TILE // CHUNK
NPAD = 10240
ROWS_PER_TILE = NPAD // NUM_SUBCORES
ZROWS = 32

_sc_mesh = plsc.VectorSubcoreMesh(core_axis_name="c", subcore_axis_name="s")


DW = 128
DCH = 80
DCHUNKS = E // NUM_SUBCORES // DCH


@functools.partial(
    pl.kernel,
    out_type=jax.ShapeDtypeStruct((NUM_CORES, NPAD, DW), jnp.float32),
    mesh=_sc_mesh,
    scratch_types=[
        pltpu.VMEM((DCHUNKS, DCH), jnp.int32),
        pltpu.VMEM((DCH, DW), jnp.float32),
        pltpu.VMEM((ZROWS, DW), jnp.float32),
        pltpu.VMEM_SHARED((NPAD, DW), jnp.float32),
        pltpu.SemaphoreType.DMA,
        [pltpu.SemaphoreType.DMA for _ in range(2)],
    ],
)
def _degree_kernel(sd_hbm, out_hbm, idx_t, ones_v, zero_v, cnt_sh, isem,
                   ssems):
    cid = lax.axis_index("c")
    sid = lax.axis_index("s")
    row0 = sid * ROWS_PER_TILE
    idx_cp = pltpu.async_copy(sd_hbm.at[cid, sid], idx_t, isem)

    nv = DW // 16

    def ofill(i, _):
        r = i // nv
        c = (i % nv) * 16
        ones_v[r, pl.ds(c, 16)] = jnp.full((16,), 1.0, jnp.float32)
        return 0

    lax.fori_loop(0, DCH * nv, ofill, 0)

    def zfill(i, _):
        r = i // nv
        c = (i % nv) * 16
        zero_v[r, pl.ds(c, 16)] = jnp.zeros((16,), jnp.float32)
        return 0

    lax.fori_loop(0, ZROWS * nv, zfill, 0)

    def zcopy(k, _):
        pltpu.sync_copy(zero_v, cnt_sh.at[pl.ds(row0 + k * ZROWS, ZROWS)])
        return 0

    lax.fori_loop(0, ROWS_PER_TILE // ZROWS, zcopy, 0)
    idx_cp.wait()
    plsc.subcore_barrier()

    def scat(c, par):
        @pl.when(c >= 2)
        def _():
            pltpu.make_async_copy(ones_v, cnt_sh.at[idx_t.at[c - 2]],
                                  ssems[par]).wait()

        pltpu.async_copy(ones_v, cnt_sh.at[idx_t.at[c]], ssems[par],
                         add=True)

    def body(i, _):
        scat(2 * i, 0)
        scat(2 * i + 1, 1)
        return 0

    lax.fori_loop(0, DCHUNKS // 2, body, 0)
    pltpu.make_async_copy(ones_v, cnt_sh.at[idx_t.at[DCHUNKS - 2]],
                          ssems[0]).wait()
    pltpu.make_async_copy(ones_v, cnt_sh.at[idx_t.at[DCHUNKS - 1]],
                          ssems[1]).wait()
    plsc.subcore_barrier()
    pltpu.sync_copy(cnt_sh.at[pl.ds(row0, ROWS_PER_TILE)],
                    out_hbm.at[cid, pl.ds(row0, ROWS_PER_TILE)])


SLOTS = 4


IDEPTH = 8


@functools.partial(
    pl.kernel,
    out_type=jax.ShapeDtypeStruct((NUM_CORES, NPAD, H), jnp.float32),
    mesh=_sc_mesh,
    scratch_types=[
        [pltpu.VMEM((CHUNK,), jnp.int32) for _ in range(IDEPTH)],
        [pltpu.VMEM((CHUNK,), jnp.int32) for _ in range(IDEPTH)],
        [pltpu.VMEM((CHUNK, H), jnp.float32) for _ in range(SLOTS)],
        pltpu.VMEM((ZROWS, H), jnp.float32),
        pltpu.VMEM_SHARED((NPAD, H), jnp.float32),
        [pltpu.SemaphoreType.DMA for _ in range(IDEPTH)],
        [pltpu.SemaphoreType.DMA for _ in range(SLOTS)],
        [pltpu.SemaphoreType.DMA for _ in range(SLOTS)],
    ],
)
def _edge_agg_kernel(p_hbm, src_hbm, dst_hbm, out_hbm, src_v, dst_v, rows,
                     zero_v, agg_sh, isems, gsems, ssems):
    cid = lax.axis_index("c")
    sid = lax.axis_index("s")
    wid = sid * NUM_CORES + cid
    row0 = sid * ROWS_PER_TILE
    base0 = wid * EDGES_PER_TILE

    def idx_issue(c, d):
        @pl.when(c < N_CHUNKS)
        def _():
            base = base0 + c * CHUNK
            pltpu.async_copy(src_hbm.at[pl.ds(base, CHUNK)], src_v[d],
                             isems[d])
            pltpu.async_copy(dst_hbm.at[pl.ds(base, CHUNK)], dst_v[d],
                             isems[d])

    def gather_issue(c, s, d):
        @pl.when(c < N_CHUNKS)
        def _():
            if_scat = c >= SLOTS

            @pl.when(if_scat)
            def _():
                pltpu.make_async_copy(rows[s], agg_sh.at[dst_v[(s + SLOTS) %
                                                               IDEPTH]],
                                      ssems[s]).wait()

            base = base0 + c * CHUNK
            pltpu.make_async_copy(src_hbm.at[pl.ds(base, CHUNK)], src_v[d],
                                  isems[d]).wait()
            pltpu.make_async_copy(dst_hbm.at[pl.ds(base, CHUNK)], dst_v[d],
                                  isems[d]).wait()
            pltpu.async_copy(p_hbm.at[src_v[d]], rows[s], gsems[s])

    def drain(c, s, d):
        @pl.when(c < N_CHUNKS)
        def _():
            pltpu.make_async_copy(p_hbm.at[src_v[d]], rows[s],
                                  gsems[s]).wait()
            pltpu.async_copy(rows[s], agg_sh.at[dst_v[d]], ssems[s],
                             add=True)

    for k in range(SLOTS):
        idx_issue(k, k)

    def zfill(i, _):
        r = i // 8
        c = (i % 8) * 16
        zero_v[r, pl.ds(c, 16)] = jnp.zeros((16,), jnp.float32)
        return 0

    lax.fori_loop(0, ZROWS * 8, zfill, 0)

    def zcopy(k, _):
        pltpu.sync_copy(zero_v, agg_sh.at[pl.ds(row0 + k * ZROWS, ZROWS)])
        return 0

    lax.fori_loop(0, ROWS_PER_TILE // ZROWS, zcopy, 0)
    plsc.subcore_barrier()

    gather_issue(0, 0, 0)
    gather_issue(1, 1, 1)

    def body(i, _):
        c0 = 8 * i
        for k in range(IDEPTH):
            c = c0 + k
            drain(c, k % SLOTS, k)
            idx_issue(c + SLOTS, (k + SLOTS) % IDEPTH)
            gather_issue(c + 2, (k + 2) % SLOTS, (k + 2) % IDEPTH)
        return 0

    lax.fori_loop(0, (N_CHUNKS + IDEPTH - 1) // IDEPTH, body, 0)

    for c in range(N_CHUNKS - SLOTS, N_CHUNKS):
        pltpu.make_async_copy(rows[c % SLOTS], agg_sh.at[dst_v[c % IDEPTH]],
                              ssems[c % SLOTS]).wait()
    plsc.subcore_barrier()
    pltpu.sync_copy(agg_sh.at[pl.ds(row0, ROWS_PER_TILE)],
                    out_hbm.at[cid, pl.ds(row0, ROWS_PER_TILE)])


BLK = 1000
GRID = N // BLK


def _setup_body(cnt, x, wr, br, wz, bz, wh, bh,
                inv_out_o, inv_in_o, xr_o, xz_o, xh_o):
    c = cnt[...]
    deg_out = c[0, :, 0]
    deg_in = c[1, :, 0]
    inv_out_o[...] = jnp.where(deg_out > 0, lax.rsqrt(deg_out), 0.0)[:, None]
    inv_in_o[...] = jnp.where(deg_in > 0, lax.rsqrt(deg_in), 0.0)[:, None]
    xr_o[...] = x[...] @ wr[...] + br[...]
    xz_o[...] = x[...] @ wz[...] + bz[...]
    xh_o[...] = x[...] @ wh[...] + bh[...]


_setup_call = pl.pallas_call(
    _setup_body,
    grid=(GRID,),
    in_specs=[
        pl.BlockSpec((2, BLK, DW), lambda i: (0, i, 0)),
        pl.BlockSpec((1, H), lambda i: (0, 0)),
        pl.BlockSpec((H, H), lambda i: (0, 0)),
        pl.BlockSpec((1, H), lambda i: (0, 0)),
        pl.BlockSpec((H, H), lambda i: (0, 0)),
        pl.BlockSpec((1, H), lambda i: (0, 0)),
        pl.BlockSpec((H, H), lambda i: (0, 0)),
        pl.BlockSpec((1, H), lambda i: (0, 0)),
    ],
    out_specs=[
        pl.BlockSpec((BLK, 1), lambda i: (i, 0)),
        pl.BlockSpec((BLK, 1), lambda i: (i, 0)),
        pl.BlockSpec((1, H), lambda i: (0, 0)),
        pl.BlockSpec((1, H), lambda i: (0, 0)),
        pl.BlockSpec((1, H), lambda i: (0, 0)),
    ],
    out_shape=[
        jax.ShapeDtypeStruct((N, 1), jnp.float32),
        jax.ShapeDtypeStruct((N, 1), jnp.float32),
        jax.ShapeDtypeStruct((1, H), jnp.float32),
        jax.ShapeDtypeStruct((1, H), jnp.float32),
        jax.ShapeDtypeStruct((1, H), jnp.float32),
    ],
)


def _step1_body(inv_out, xr, xz, xh, gb, gw, h_o, p_o):
    gh = gb[...]
    r = jax.nn.sigmoid(xr[...] + gh)
    z = jax.nn.sigmoid(xz[...] + gh)
    ht = jnp.tanh(xh[...] + r * gh)
    h1 = z * ht
    h_o[...] = jnp.broadcast_to(h1, (BLK, H))
    p_o[...] = inv_out[...] * jnp.dot(h1, gw[...],
                                      preferred_element_type=jnp.float32)


_step1_call = pl.pallas_call(
    _step1_body,
    grid=(GRID,),
    in_specs=[
        pl.BlockSpec((BLK, 1), lambda i: (i, 0)),
        pl.BlockSpec((1, H), lambda i: (0, 0)),
        pl.BlockSpec((1, H), lambda i: (0, 0)),
        pl.BlockSpec((1, H), lambda i: (0, 0)),
        pl.BlockSpec((1, H), lambda i: (0, 0)),
        pl.BlockSpec((H, H), lambda i: (0, 0)),
    ],
    out_specs=[
        pl.BlockSpec((BLK, H), lambda i: (i, 0)),
        pl.BlockSpec((BLK, H), lambda i: (i, 0)),
    ],
    out_shape=[
        jax.ShapeDtypeStruct((N, H), jnp.float32),
        jax.ShapeDtypeStruct((N, H), jnp.float32),
    ],
)


def _step_body(h, agg, inv_in, inv_out, xr, xz, xh, gb, gw, h_o, p_o):
    a = agg[...]
    gh = (a[0] + a[1]) * inv_in[...] + gb[...]
    r = jax.nn.sigmoid(xr[...] + gh)
    z = jax.nn.sigmoid(xz[...] + gh)
    ht = jnp.tanh(xh[...] + r * gh)
    hn = (1.0 - z) * h[...] + z * ht
    h_o[...] = hn
    p_o[...] = jnp.dot(hn * inv_out[...], gw[...],
                       preferred_element_type=jnp.float32)


_step_call = pl.pallas_call(
    _step_body,
    grid=(GRID,),
    in_specs=[
        pl.BlockSpec((BLK, H), lambda i: (i, 0)),
        pl.BlockSpec((2, BLK, H), lambda i: (0, i, 0)),
        pl.BlockSpec((BLK, 1), lambda i: (i, 0)),
        pl.BlockSpec((BLK, 1), lambda i: (i, 0)),
        pl.BlockSpec((1, H), lambda i: (0, 0)),
        pl.BlockSpec((1, H), lambda i: (0, 0)),
        pl.BlockSpec((1, H), lambda i: (0, 0)),
        pl.BlockSpec((1, H), lambda i: (0, 0)),
        pl.BlockSpec((H, H), lambda i: (0, 0)),
    ],
    out_specs=[
        pl.BlockSpec((BLK, H), lambda i: (i, 0)),
        pl.BlockSpec((BLK, H), lambda i: (i, 0)),
    ],
    out_shape=[
        jax.ShapeDtypeStruct((N, H), jnp.float32),
        jax.ShapeDtypeStruct((N, H), jnp.float32),
    ],
)


def kernel(x, edge_index, w_r_W, w_r_b, w_z_W, w_z_b, w_h_W, w_h_b, gcn_W,
           gcn_b):
    x2 = x.reshape(1, H)
    br = w_r_b.reshape(1, H)
    bz = w_z_b.reshape(1, H)
    bh = w_h_b.reshape(1, H)
    gb = gcn_b.reshape(1, H)

    src = edge_index[0]
    dst = edge_index[1]
    sd = edge_index.reshape(2, NUM_SUBCORES, DCHUNKS, DCH)
    cnt = _degree_kernel(sd)
    inv_out, inv_in, xr, xz, xh = _setup_call(
        cnt, x2, w_r_W, br, w_z_W, bz, w_h_W, bh)

    h, p = _step1_call(inv_out, xr, xz, xh, gb, gcn_W)
    outs = [h]
    for _ in range(1, SEQ):
        agg = _edge_agg_kernel(p, src, dst)
        h, p = _step_call(h, agg, inv_in, inv_out, xr, xz, xh, gb, gcn_W)
        outs.append(h)
    return jnp.stack(outs, axis=0)[None]
